# Initial kernel scaffold; baseline (speedup 1.0000x reference)
import jax
import jax.numpy as jnp
from jax.experimental import pallas as pl


def kernel(x, edge_index, W_msg, b_msg, W_upd, b_upd):
    raise NotImplementedError("write your pallas kernel here")


# speedup vs baseline: 5.8273x; 5.8273x over previous
import functools

import jax
import jax.numpy as jnp
from jax import lax
from jax.experimental import pallas as pl
from jax.experimental.pallas import tpu as pltpu
from jax.experimental.pallas import tpu_sc as plsc

NC = 2
NS = 16
L = 16

CH = 80


def _pre_body(x_ref, w1_ref, w2_ref, b_ref, p_ref, q_ref):
    x = x_ref[...]
    p_ref[...] = jnp.dot(x, w1_ref[...], preferred_element_type=jnp.float32)
    q_ref[...] = (
        jnp.dot(x, w2_ref[...], preferred_element_type=jnp.float32)
        + b_ref[...]
    )


def _upd_body(agg_ref, x_ref, w1_ref, w2_ref, b_ref, o_ref):
    n = x_ref.shape[0]
    a = agg_ref[0, :n] + agg_ref[1, :n]
    o_ref[...] = jnp.maximum(
        jnp.dot(a, w1_ref[...], preferred_element_type=jnp.float32)
        + jnp.dot(x_ref[...], w2_ref[...], preferred_element_type=jnp.float32)
        + b_ref[...],
        0.0,
    )


def _make_sc_edge(N, D, E):
    assert E % (NC * NS) == 0
    ew = E // (NC * NS)
    assert ew % CH == 0
    n_chunks = ew // CH
    n_pad = -(-N // (NS * 32)) * (NS * 32)
    rows_w = n_pad // NS
    st = rows_w // 4
    n_st = 4
    assert st % 8 == 0

    mesh = plsc.VectorSubcoreMesh(
        core_axis_name="c", subcore_axis_name="s",
        num_cores=NC, num_subcores=NS,
    )

    @functools.partial(
        pl.kernel,
        out_type=jax.ShapeDtypeStruct((NC, n_pad, D), jnp.float32),
        mesh=mesh,
        scratch_types=[
            pltpu.VMEM((CH,), jnp.int32),
            pltpu.VMEM((CH,), jnp.int32),
            pltpu.VMEM((CH, D), jnp.float32),
            pltpu.VMEM((CH, D), jnp.float32),
            pltpu.VMEM((st, D), jnp.float32),
            pltpu.VMEM_SHARED((n_pad, D), jnp.float32),
            pltpu.SemaphoreType.DMA,
            pltpu.SemaphoreType.DMA,
        ],
    )
    def sc_edge(p_hbm, q_hbm, src_hbm, dst_hbm, out_hbm,
                sidx, didx, prow, qrow, stage, agg_sh, sem1, sem2):
        cid = lax.axis_index("c")
        sid = lax.axis_index("s")

        def zrow(r, _):
            for k in range(D // L):
                stage[r, pl.ds(k * L, L)] = jnp.zeros((L,), jnp.float32)
            return 0
        lax.fori_loop(0, st, zrow, 0)
        for j in range(n_st):
            pltpu.sync_copy(stage, agg_sh.at[pl.ds(sid * rows_w + j * st, st)])
        plsc.subcore_barrier()

        base_w = (cid * NS + sid) * ew

        def body(i, _):
            base = base_w + i * CH
            pltpu.sync_copy(src_hbm.at[pl.ds(base, CH)], sidx)
            pltpu.sync_copy(dst_hbm.at[pl.ds(base, CH)], didx)
            cp1 = pltpu.async_copy(p_hbm.at[sidx], prow, sem1)
            cp2 = pltpu.async_copy(q_hbm.at[didx], qrow, sem2)
            cp1.wait()
            cp2.wait()

            def rbody(r, _):
                for k in range(D // L):
                    s = pl.ds(k * L, L)
                    prow[r, s] = jnp.maximum(prow[r, s] + qrow[r, s], 0.0)
                return 0
            lax.fori_loop(0, CH, rbody, 0)

            pltpu.sync_copy(prow, agg_sh.at[didx], add=True)
            return 0
        lax.fori_loop(0, n_chunks, body, 0)

        plsc.subcore_barrier()
        for j in range(n_st):
            off = sid * rows_w + j * st
            pltpu.sync_copy(agg_sh.at[pl.ds(off, st)], stage)
            pltpu.sync_copy(stage, out_hbm.at[cid, pl.ds(off, st)])

    return sc_edge


def kernel(x, edge_index, W_msg, b_msg, W_upd, b_upd):
    N, D = x.shape
    E = edge_index.shape[1]

    P, Q = pl.pallas_call(
        _pre_body,
        out_shape=[jax.ShapeDtypeStruct((N, D), jnp.float32)] * 2,
    )(x, W_msg[:D], W_msg[D:], b_msg.reshape(1, D))

    agg = _make_sc_edge(N, D, E)(P, Q, edge_index[0], edge_index[1])

    out = pl.pallas_call(
        _upd_body,
        out_shape=jax.ShapeDtypeStruct((N, D), jnp.float32),
    )(agg, x, W_upd[:D], W_upd[D:], b_upd.reshape(1, D))
    return out


# speedup vs baseline: 8.8074x; 1.5114x over previous
import functools

import jax
import jax.numpy as jnp
from jax import lax
from jax.experimental import pallas as pl
from jax.experimental.pallas import tpu as pltpu
from jax.experimental.pallas import tpu_sc as plsc

NC = 2
NS = 16
L = 16

CH = 80


def _pre_body(x_ref, w1_ref, w2_ref, b_ref, p_ref, q_ref):
    x = x_ref[...]
    p_ref[...] = jnp.dot(x, w1_ref[...], preferred_element_type=jnp.float32)
    q_ref[...] = (
        jnp.dot(x, w2_ref[...], preferred_element_type=jnp.float32)
        + b_ref[...]
    )


def _upd_body(agg_ref, x_ref, w1_ref, w2_ref, b_ref, o_ref):
    n = x_ref.shape[0]
    a = agg_ref[0, :n] + agg_ref[1, :n]
    o_ref[...] = jnp.maximum(
        jnp.dot(a, w1_ref[...], preferred_element_type=jnp.float32)
        + jnp.dot(x_ref[...], w2_ref[...], preferred_element_type=jnp.float32)
        + b_ref[...],
        0.0,
    )


def _make_sc_edge(N, D, E):
    assert E % (NC * NS) == 0
    ew = E // (NC * NS)
    assert ew % CH == 0
    n_chunks = ew // CH
    n_pad = -(-N // (NS * CH)) * (NS * CH)
    rows_w = n_pad // NS
    st = CH
    n_st = rows_w // st
    assert st % 8 == 0 and rows_w % st == 0

    mesh = plsc.VectorSubcoreMesh(
        core_axis_name="c", subcore_axis_name="s",
        num_cores=NC, num_subcores=NS,
    )

    assert n_chunks >= 3 and n_chunks % 2 == 1
    kk = (n_chunks - 3) // 2

    @functools.partial(
        pl.kernel,
        out_type=jax.ShapeDtypeStruct((NC, n_pad, D), jnp.float32),
        mesh=mesh,
        scratch_types=[
            pltpu.VMEM((CH,), jnp.int32),
            pltpu.VMEM((CH,), jnp.int32),
            pltpu.VMEM((CH,), jnp.int32),
            pltpu.VMEM((CH,), jnp.int32),
            pltpu.VMEM((CH, D), jnp.float32),
            pltpu.VMEM((CH, D), jnp.float32),
            pltpu.VMEM((CH, D), jnp.float32),
            pltpu.VMEM((CH, D), jnp.float32),
            pltpu.VMEM_SHARED((n_pad, D), jnp.float32),
            pltpu.SemaphoreType.DMA,
            pltpu.SemaphoreType.DMA,
        ],
    )
    def sc_edge(p_hbm, q_hbm, src_hbm, dst_hbm, out_hbm,
                sidx0, sidx1, didx0, didx1, prow0, prow1, qrow0, qrow1,
                agg_sh, semg, sems):
        cid = lax.axis_index("c")
        sid = lax.axis_index("s")
        sx = [sidx0, sidx1]
        dx = [didx0, didx1]
        pr = [prow0, prow1]
        qr = [qrow0, qrow1]

        def zrow(r, _):
            for k in range(D // L):
                prow0[r, pl.ds(k * L, L)] = jnp.zeros((L,), jnp.float32)
            return 0
        lax.fori_loop(0, st, zrow, 0)
        for j in range(n_st):
            pltpu.sync_copy(prow0, agg_sh.at[pl.ds(sid * rows_w + j * st, st)])
        plsc.subcore_barrier()

        base_w = (cid * NS + sid) * ew

        def load_idx(i, b):
            base = base_w + i * CH
            pltpu.sync_copy(src_hbm.at[pl.ds(base, CH)], sx[b])
            pltpu.sync_copy(dst_hbm.at[pl.ds(base, CH)], dx[b])

        def fire_gather(b):
            pltpu.async_copy(p_hbm.at[sx[b]], pr[b], semg)
            pltpu.async_copy(q_hbm.at[dx[b]], qr[b], semg)

        def wait_gather(b):
            pltpu.make_async_copy(p_hbm.at[sx[b]], pr[b], semg).wait()
            pltpu.make_async_copy(q_hbm.at[dx[b]], qr[b], semg).wait()

        def fire_scatter(b):
            pltpu.async_copy(pr[b], agg_sh.at[dx[b]], sems, add=True)

        def wait_scatter(b):
            pltpu.make_async_copy(pr[b], agg_sh.at[dx[b]], sems).wait()

        def compute(b):
            @plsc.parallel_loop(0, CH, unroll=2)
            def _(r):
                for k in range(D // L):
                    s = pl.ds(k * L, L)
                    pr[b][r, s] = jnp.maximum(pr[b][r, s] + qr[b][r, s], 0.0)

        def full_step(i, b, prefetch, pending_scatter):
            if prefetch:
                if pending_scatter:
                    wait_scatter(1 - b)
                load_idx(i + 1, 1 - b)
                fire_gather(1 - b)
            wait_gather(b)
            compute(b)
            fire_scatter(b)

        load_idx(0, 0)
        fire_gather(0)
        full_step(0, 0, prefetch=True, pending_scatter=False)

        def body(j, _):
            i = 2 * j + 1
            full_step(i, 1, prefetch=True, pending_scatter=True)
            full_step(i + 1, 0, prefetch=True, pending_scatter=True)
            return 0
        lax.fori_loop(0, kk, body, 0)

        full_step(n_chunks - 2, 1, prefetch=True, pending_scatter=True)
        full_step(n_chunks - 1, 0, prefetch=False, pending_scatter=False)
        wait_scatter(1)
        wait_scatter(0)

        plsc.subcore_barrier()
        for j in range(n_st):
            off = sid * rows_w + j * st
            pltpu.sync_copy(agg_sh.at[pl.ds(off, st)], prow0)
            pltpu.sync_copy(prow0, out_hbm.at[cid, pl.ds(off, st)])

    return sc_edge


def kernel(x, edge_index, W_msg, b_msg, W_upd, b_upd):
    N, D = x.shape
    E = edge_index.shape[1]

    P, Q = pl.pallas_call(
        _pre_body,
        out_shape=[jax.ShapeDtypeStruct((N, D), jnp.float32)] * 2,
    )(x, W_msg[:D], W_msg[D:], b_msg.reshape(1, D))

    agg = _make_sc_edge(N, D, E)(P, Q, edge_index[0], edge_index[1])

    out = pl.pallas_call(
        _upd_body,
        out_shape=jax.ShapeDtypeStruct((N, D), jnp.float32),
    )(agg, x, W_upd[:D], W_upd[D:], b_upd.reshape(1, D))
    return out


# speedup vs baseline: 11.0694x; 1.2568x over previous
import functools

import jax
import jax.numpy as jnp
from jax import lax
from jax.experimental import pallas as pl
from jax.experimental.pallas import tpu as pltpu
from jax.experimental.pallas import tpu_sc as plsc

NC = 2
NS = 16
L = 16

CH = 80


def _pre_body(x_ref, w1_ref, w2_ref, b_ref, p_ref, q_ref):
    x = x_ref[...]
    p_ref[...] = jnp.dot(x, w1_ref[...], preferred_element_type=jnp.float32)
    q_ref[...] = (
        jnp.dot(x, w2_ref[...], preferred_element_type=jnp.float32)
        + b_ref[...]
    )


def _upd_body(agg_ref, x_ref, w1_ref, w2_ref, b_ref, o_ref):
    n = x_ref.shape[0]
    a = agg_ref[0, :n] + agg_ref[1, :n]
    o_ref[...] = jnp.maximum(
        jnp.dot(a, w1_ref[...], preferred_element_type=jnp.float32)
        + jnp.dot(x_ref[...], w2_ref[...], preferred_element_type=jnp.float32)
        + b_ref[...],
        0.0,
    )


def _make_sc_edge(N, D, E):
    assert E % (NC * NS) == 0
    ew = E // (NC * NS)
    assert ew % CH == 0
    n_chunks = ew // CH
    n_pad = -(-N // (NS * CH)) * (NS * CH)
    rows_w = n_pad // NS
    st = CH
    n_st = rows_w // st
    assert st % 8 == 0 and rows_w % st == 0

    mesh = plsc.VectorSubcoreMesh(
        core_axis_name="c", subcore_axis_name="s",
        num_cores=NC, num_subcores=NS,
    )

    SBC = 25
    assert n_chunks % SBC == 0 and SBC % 2 == 1 and SBC >= 3
    n_sup = n_chunks // SBC
    kk = (SBC - 3) // 2

    @functools.partial(
        pl.kernel,
        out_type=jax.ShapeDtypeStruct((NC, n_pad, D), jnp.float32),
        mesh=mesh,
        scratch_types=[
            pltpu.VMEM((SBC * CH,), jnp.int32),
            pltpu.VMEM((SBC * CH,), jnp.int32),
            pltpu.VMEM((CH,), jnp.int32),
            pltpu.VMEM((CH,), jnp.int32),
            pltpu.VMEM((CH, D), jnp.float32),
            pltpu.VMEM((CH, D), jnp.float32),
            pltpu.VMEM((CH, D), jnp.float32),
            pltpu.VMEM((CH, D), jnp.float32),
            pltpu.VMEM_SHARED((n_pad, D), jnp.float32),
            pltpu.SemaphoreType.DMA,
            pltpu.SemaphoreType.DMA,
        ],
    )
    def sc_edge(p_hbm, q_hbm, src_hbm, dst_hbm, out_hbm,
                sidx_sc, didx_sc, didx0, didx1, prow0, prow1, qrow0, qrow1,
                agg_sh, semg, sems):
        cid = lax.axis_index("c")
        sid = lax.axis_index("s")
        dxb = [didx0, didx1]
        pr = [prow0, prow1]
        qr = [qrow0, qrow1]

        def zrow(r, _):
            for k in range(D // L):
                prow0[r, pl.ds(k * L, L)] = jnp.zeros((L,), jnp.float32)
            return 0
        lax.fori_loop(0, st, zrow, 0)
        for j in range(n_st):
            pltpu.sync_copy(prow0, agg_sh.at[pl.ds(sid * rows_w + j * st, st)])
        plsc.subcore_barrier()

        base_w = (cid * NS + sid) * ew

        def load_super(s):
            base = base_w + s * (SBC * CH)
            pltpu.sync_copy(src_hbm.at[pl.ds(base, SBC * CH)], sidx_sc)
            pltpu.sync_copy(dst_hbm.at[pl.ds(base, SBC * CH)], didx_sc)

        def fire_gather(c, b):
            pltpu.async_copy(
                p_hbm.at[sidx_sc.at[pl.ds(c * CH, CH)]], pr[b], semg)
            pltpu.async_copy(
                q_hbm.at[didx_sc.at[pl.ds(c * CH, CH)]], qr[b], semg)

        def wait_gather(b):
            pltpu.make_async_copy(
                p_hbm.at[sidx_sc.at[pl.ds(0, CH)]], pr[b], semg).wait()
            pltpu.make_async_copy(
                q_hbm.at[didx_sc.at[pl.ds(0, CH)]], qr[b], semg).wait()

        def fire_scatter(b):
            pltpu.async_copy(pr[b], agg_sh.at[dxb[b]], sems, add=True)

        def wait_scatter(b):
            pltpu.make_async_copy(pr[b], agg_sh.at[dxb[b]], sems).wait()

        def stage_didx(c, b):
            for k in range(CH // L):
                dxb[b][pl.ds(k * L, L)] = didx_sc[pl.ds(c * CH + k * L, L)]

        def compute(b):
            @plsc.parallel_loop(0, CH, unroll=2)
            def _(r):
                for k in range(D // L):
                    s = pl.ds(k * L, L)
                    pr[b][r, s] = jnp.maximum(pr[b][r, s] + qr[b][r, s], 0.0)

        def full_step(c, b, prefetch, pending_scatter):
            if prefetch:
                if pending_scatter:
                    wait_scatter(1 - b)
                fire_gather(c + 1, 1 - b)
            wait_gather(b)
            stage_didx(c, b)
            compute(b)
            fire_scatter(b)

        def super_body(s, first):
            load_super(s)
            if not first:
                wait_scatter(0)
            fire_gather(0, 0)
            full_step(0, 0, prefetch=True, pending_scatter=not first)

            def body(j, _):
                c = 2 * j + 1
                full_step(c, 1, prefetch=True, pending_scatter=True)
                full_step(c + 1, 0, prefetch=True, pending_scatter=True)
                return 0
            lax.fori_loop(0, kk, body, 0)

            full_step(SBC - 2, 1, prefetch=True, pending_scatter=True)
            full_step(SBC - 1, 0, prefetch=False, pending_scatter=False)

        super_body(0, first=True)

        def sbody(s, _):
            super_body(s, first=False)
            return 0
        lax.fori_loop(1, n_sup, sbody, 0)
        wait_scatter(1)
        wait_scatter(0)

        plsc.subcore_barrier()
        for j in range(n_st):
            off = sid * rows_w + j * st
            pltpu.sync_copy(agg_sh.at[pl.ds(off, st)], prow0)
            pltpu.sync_copy(prow0, out_hbm.at[cid, pl.ds(off, st)])

    return sc_edge


def kernel(x, edge_index, W_msg, b_msg, W_upd, b_upd):
    N, D = x.shape
    E = edge_index.shape[1]

    P, Q = pl.pallas_call(
        _pre_body,
        out_shape=[jax.ShapeDtypeStruct((N, D), jnp.float32)] * 2,
    )(x, W_msg[:D], W_msg[D:], b_msg.reshape(1, D))

    agg = _make_sc_edge(N, D, E)(P, Q, edge_index[0], edge_index[1])

    out = pl.pallas_call(
        _upd_body,
        out_shape=jax.ShapeDtypeStruct((N, D), jnp.float32),
    )(agg, x, W_upd[:D], W_upd[D:], b_upd.reshape(1, D))
    return out
